# baseline (speedup 1.0000x reference)
import jax
import jax.numpy as jnp
from jax.experimental import pallas as pl

P = 64
Q = 1024
HID = 64
OUT = 16
NEG = 0.01
PELLET_MASS = 0.01


def _lrelu(x):
    return jnp.where(x >= 0, x, NEG * x)


def _gnn_body(masses, locs, pellets,
              Wpp0, bpp0, Wl0, bl0, Wr0, Wqq0, bqq0,
              Wpp1, bpp1, Wl1, bl1, Wr1,
              Wpost, bpost, out_ref):
    m = masses[:, :]
    L = locs[:, :]

    mass_mean = jnp.mean(m)
    lmean = jnp.mean(L, axis=0, keepdims=True)
    pel = pellets[:, :]
    odd = jax.lax.broadcasted_iota(jnp.int32, (8, 256), 1) % 2
    pm0 = jnp.sum(jnp.where(odd == 0, pel, 0.0)) * (1.0 / Q)
    pm1 = jnp.sum(jnp.where(odd == 0, 0.0, pel)) * (1.0 / Q)
    lm0 = lmean[0, 0]
    lm1 = lmean[0, 1]

    c0 = (mass_mean * Wpp0[0:1, :] + lm0 * Wpp0[1:2, :] + lm1 * Wpp0[2:3, :]
          + bpp0[:, :]
          + PELLET_MASS * Wl0[0:1, :] + pm0 * Wl0[1:2, :] + pm1 * Wl0[2:3, :]
          + bl0[:, :])
    h0 = (m * Wr0[0:1, :] + L[:, 0:1] * Wr0[1:2, :] + L[:, 1:2] * Wr0[2:3, :]
          + c0)
    x_p1 = _lrelu(h0)
    q1 = _lrelu(PELLET_MASS * Wqq0[0:1, :] + pm0 * Wqq0[1:2, :]
                + pm1 * Wqq0[2:3, :] + bqq0[:, :])

    m_p1 = jnp.mean(x_p1, axis=0, keepdims=True)
    c1 = (jnp.dot(m_p1, Wpp1[:, :], preferred_element_type=jnp.float32)
          + bpp1[:, :]
          + jnp.dot(q1, Wl1[:, :], preferred_element_type=jnp.float32)
          + bl1[:, :])
    x_p2 = _lrelu(jnp.dot(x_p1, Wr1[:, :], preferred_element_type=jnp.float32)
                  + c1)

    logits = (jnp.dot(x_p2, Wpost[:, :], preferred_element_type=jnp.float32)
              + bpost[:, :])
    mx = jnp.max(logits, axis=-1, keepdims=True)
    s = logits - mx
    out_ref[:, :] = s - jnp.log(jnp.sum(jnp.exp(s), axis=-1, keepdims=True))


def kernel(player_masses, player_locations, pellet_locations,
           W_gcn_pp0, b_gcn_pp0, W_sage_l0, b_sage_l0, W_sage_r0, W_gcn_qq0, b_gcn_qq0,
           W_gcn_pp1, b_gcn_pp1, W_sage_l1, b_sage_l1, W_sage_r1, W_gcn_qq1, b_gcn_qq1,
           W_post, b_post):
    args = (
        player_masses.reshape(P, 1),
        player_locations,
        pellet_locations.reshape(8, 256),
        W_gcn_pp0, b_gcn_pp0.reshape(1, HID),
        W_sage_l0, b_sage_l0.reshape(1, HID), W_sage_r0,
        W_gcn_qq0, b_gcn_qq0.reshape(1, HID),
        W_gcn_pp1, b_gcn_pp1.reshape(1, HID),
        W_sage_l1, b_sage_l1.reshape(1, HID), W_sage_r1,
        W_post, b_post.reshape(1, OUT),
    )
    return pl.pallas_call(
        _gnn_body,
        out_shape=jax.ShapeDtypeStruct((P, OUT), jnp.float32),
    )(*args)


# speedup vs baseline: 1.2790x; 1.2790x over previous
import jax
import jax.numpy as jnp
from jax.experimental import pallas as pl

P = 64
Q = 1024
HID = 64
OUT = 16
NEG = 0.01
PELLET_MASS = 0.01


def _lrelu(x):
    return jnp.where(x >= 0, x, NEG * x)


def _gnn_body(masses, locs, pellets,
              Wpp0, bpp0, Wl0, bl0, Wr0, Wqq0, bqq0,
              Wpp1, bpp1, Wl1, bl1, Wr1,
              Wpost, bpost, out_ref):
    m = masses[:, :]
    L = locs[:, :]

    mass_mean = jnp.mean(m)
    lmean = jnp.mean(L, axis=0, keepdims=True)
    pmean = jnp.mean(pellets[:, :], axis=1, keepdims=True)
    lm0 = lmean[0, 0]
    lm1 = lmean[0, 1]
    pm0 = pmean[0, 0]
    pm1 = pmean[1, 0]

    c0 = (mass_mean * Wpp0[0:1, :] + lm0 * Wpp0[1:2, :] + lm1 * Wpp0[2:3, :]
          + bpp0[:, :]
          + PELLET_MASS * Wl0[0:1, :] + pm0 * Wl0[1:2, :] + pm1 * Wl0[2:3, :]
          + bl0[:, :])
    h0 = (m * Wr0[0:1, :] + L[:, 0:1] * Wr0[1:2, :] + L[:, 1:2] * Wr0[2:3, :]
          + c0)
    x_p1 = _lrelu(h0)
    q1 = _lrelu(PELLET_MASS * Wqq0[0:1, :] + pm0 * Wqq0[1:2, :]
                + pm1 * Wqq0[2:3, :] + bqq0[:, :])

    m_p1 = jnp.mean(x_p1, axis=0, keepdims=True)
    c1 = (jnp.dot(m_p1, Wpp1[:, :], preferred_element_type=jnp.float32)
          + bpp1[:, :]
          + jnp.dot(q1, Wl1[:, :], preferred_element_type=jnp.float32)
          + bl1[:, :])
    x_p2 = _lrelu(jnp.dot(x_p1, Wr1[:, :], preferred_element_type=jnp.float32)
                  + c1)

    logits = (jnp.dot(x_p2, Wpost[:, :], preferred_element_type=jnp.float32)
              + bpost[:, :])
    mx = jnp.max(logits, axis=-1, keepdims=True)
    s = logits - mx
    out_ref[:, :] = s - jnp.log(jnp.sum(jnp.exp(s), axis=-1, keepdims=True))


def kernel(player_masses, player_locations, pellet_locations,
           W_gcn_pp0, b_gcn_pp0, W_sage_l0, b_sage_l0, W_sage_r0, W_gcn_qq0, b_gcn_qq0,
           W_gcn_pp1, b_gcn_pp1, W_sage_l1, b_sage_l1, W_sage_r1, W_gcn_qq1, b_gcn_qq1,
           W_post, b_post):
    args = (
        player_masses.reshape(P, 1),
        player_locations,
        pellet_locations.T,
        W_gcn_pp0, b_gcn_pp0.reshape(1, HID),
        W_sage_l0, b_sage_l0.reshape(1, HID), W_sage_r0,
        W_gcn_qq0, b_gcn_qq0.reshape(1, HID),
        W_gcn_pp1, b_gcn_pp1.reshape(1, HID),
        W_sage_l1, b_sage_l1.reshape(1, HID), W_sage_r1,
        W_post, b_post.reshape(1, OUT),
    )
    return pl.pallas_call(
        _gnn_body,
        out_shape=jax.ShapeDtypeStruct((P, OUT), jnp.float32),
    )(*args)
